# baseline (speedup 1.0000x reference)
import functools

import jax
import jax.numpy as jnp
from jax import lax
from jax.experimental import pallas as pl
from jax.experimental.pallas import tpu as pltpu
from jax.experimental.pallas import tpu_sc as plsc

_B, _C, _H, _W = 4, 192, 14, 14
_N = _H * _W
_T = _B * _N
_D = 32
_K = 1024
_TP = 1024


def _tc1(x_ref, sp_ref, wd_ref, bd_ref,
         xn_ref, idx_ref, em_ref, me_ref, el_ref):
    xall = jnp.concatenate([x_ref[b] for b in range(_B)], axis=1)
    y = jnp.dot(wd_ref[...], xall,
                preferred_element_type=jnp.float32)
    y = y + bd_ref[...].reshape(_D, 1)
    norms = jnp.sqrt(jnp.sum(y * y, axis=0, keepdims=True))
    xn = y / norms
    xn_ref[...] = xn

    dots = jnp.dot(sp_ref[...], xn, precision=jax.lax.Precision.HIGHEST,
                   preferred_element_type=jnp.float32)
    dmax = jnp.max(dots, axis=0, keepdims=True)
    kiota = jax.lax.broadcasted_iota(jnp.int32, (_K, _T), 0)
    idx_ref[:, :_T] = jnp.min(
        jnp.where(dots == dmax, kiota, jnp.int32(1 << 30)),
        axis=0, keepdims=True)
    idx_ref[:, _T:] = jnp.zeros((1, _TP - _T), jnp.int32)

    z = (dots - dmax) * 100.0
    ex = jnp.exp(z)
    sums = jnp.sum(ex, axis=0, keepdims=True)
    inv = 1.0 / sums
    sez = jnp.sum(ex * z, axis=0, keepdims=True)
    sample_entropy = jnp.log(sums) - inv * sez
    em = jnp.sum(sample_entropy, axis=1, keepdims=True) / float(_T)
    ap = jnp.sum(ex * inv, axis=1, keepdims=True) / float(_T)
    me = -jnp.sum(ap * jnp.log(ap + 1e-15), axis=0, keepdims=True)
    em_ref[...] = em
    me_ref[...] = me
    el_ref[...] = em - me


def _make_sc_gather():
    info = plsc.get_sparse_core_info()
    nw = info.num_cores * info.num_subcores
    b_per_w = _TP // nw
    mesh = plsc.VectorSubcoreMesh(core_axis_name="c", subcore_axis_name="s")

    @functools.partial(
        pl.kernel, mesh=mesh,
        compiler_params=pltpu.CompilerParams(use_tc_tiling_on_sc=False),
        out_type=jax.ShapeDtypeStruct((_TP, _D), jnp.float32),
        scratch_types=[
            pltpu.VMEM((b_per_w,), jnp.int32),
            pltpu.VMEM((b_per_w, _D), jnp.float32),
            pltpu.SemaphoreType.DMA,
        ],
    )
    def gather(table_hbm, idx_hbm, out_hbm, idx_v, rows_v, sem):
        wid = lax.axis_index("s") * info.num_cores + lax.axis_index("c")
        base = wid * b_per_w
        pltpu.sync_copy(idx_hbm.at[pl.ds(base, b_per_w)], idx_v)
        pltpu.async_copy(table_hbm.at[idx_v], rows_v, sem).wait()
        pltpu.sync_copy(rows_v, out_hbm.at[pl.ds(base, b_per_w)])

    return gather


_sc_gather = _make_sc_gather()


def _tc2(xn_ref, rows_ref, wu_ref, bu_ref, q_ref, cl_ref):
    quant = rows_ref[: _T].T
    xn = xn_ref[...]
    st = xn + (quant - xn)
    diff = xn - st
    cl_ref[...] = jnp.sum(jnp.sum(diff * diff, axis=0, keepdims=True),
                          axis=1, keepdims=True) / float(_D * _T)
    q = jnp.dot(wu_ref[...], st,
                preferred_element_type=jnp.float32)
    q = q + bu_ref[...].reshape(_C, 1)
    for b in range(_B):
        q_ref[b] = q[:, b * _N:(b + 1) * _N]


def kernel(x, super_points, W_down, b_down, W_up, b_up):
    x3 = x.reshape(_B, _C, _N)
    xn, idx, em, me, el = pl.pallas_call(
        _tc1,
        out_shape=(
            jax.ShapeDtypeStruct((_D, _T), jnp.float32),
            jax.ShapeDtypeStruct((1, _TP), jnp.int32),
            jax.ShapeDtypeStruct((1, 1), jnp.float32),
            jax.ShapeDtypeStruct((1, 1), jnp.float32),
            jax.ShapeDtypeStruct((1, 1), jnp.float32),
        ),
    )(x3, super_points, W_down, b_down)

    idx_flat = idx[0, :_T]
    rows = _sc_gather(super_points, idx.reshape(-1))

    q3, cl = pl.pallas_call(
        _tc2,
        out_shape=(
            jax.ShapeDtypeStruct((_B, _C, _N), jnp.float32),
            jax.ShapeDtypeStruct((1, 1), jnp.float32),
        ),
    )(xn, rows, W_up, b_up)

    q = q3.reshape(_B, _C, _H, _W)
    return (q, idx_flat, em.reshape(()), me.reshape(()),
            el.reshape(()), cl.reshape(()))


# speedup vs baseline: 2.2758x; 2.2758x over previous
import jax
import jax.numpy as jnp
from jax.experimental import pallas as pl

_B, _C, _H, _W = 4, 192, 14, 14
_N = _H * _W
_T = _B * _N
_D = 32
_K = 1024


def _body(x_ref, sp_ref, wd_ref, bd_ref, wu_ref, bu_ref,
          q_ref, idx_ref, em_ref, me_ref, el_ref, cl_ref):
    wd = wd_ref[...]
    y = jnp.concatenate(
        [jnp.dot(wd, x_ref[b], preferred_element_type=jnp.float32)
         for b in range(_B)], axis=1)
    y = y + bd_ref[...].reshape(_D, 1)

    norms = jnp.sqrt(jnp.sum(y * y, axis=0, keepdims=True))
    xn = y / norms

    sp = sp_ref[...]
    dots = jnp.dot(sp, xn, precision=jax.lax.Precision.HIGHEST,
                   preferred_element_type=jnp.float32)
    dmax = jnp.max(dots, axis=0, keepdims=True)
    kiota = jax.lax.broadcasted_iota(jnp.int32, (_K, _T), 0)
    idx = jnp.min(jnp.where(dots == dmax, kiota, jnp.int32(1 << 30)),
                  axis=0, keepdims=True)
    idx_ref[...] = idx

    z = (dots - dmax) * 100.0
    ex = jnp.exp(z)
    sums = jnp.sum(ex, axis=0, keepdims=True)
    inv = 1.0 / sums
    sez = jnp.sum(ex * z, axis=0, keepdims=True)
    sample_entropy = jnp.log(sums) - inv * sez
    em = jnp.sum(sample_entropy, axis=1, keepdims=True) / float(_T)
    ap = jnp.sum(ex * inv, axis=1, keepdims=True) / float(_T)
    me = -jnp.sum(ap * jnp.log(ap + 1e-15), axis=0, keepdims=True)

    oh = (kiota == idx).astype(jnp.float32)
    quant = jax.lax.dot_general(
        sp, oh, (((0,), (0,)), ((), ())),
        preferred_element_type=jnp.float32)

    st = xn + (quant - xn)
    diff = xn - st
    cl = jnp.sum(jnp.sum(diff * diff, axis=0, keepdims=True),
                 axis=1, keepdims=True) / float(_D * _T)

    em_ref[...] = em
    me_ref[...] = me
    el_ref[...] = em - me
    cl_ref[...] = cl

    q = jnp.dot(wu_ref[...], st,
                preferred_element_type=jnp.float32)
    q = q + bu_ref[...].reshape(_C, 1)
    for b in range(_B):
        q_ref[b] = q[:, b * _N:(b + 1) * _N]


def kernel(x, super_points, W_down, b_down, W_up, b_up):
    x3 = x.reshape(_B, _C, _N)
    q3, idx, em, me, el, cl = pl.pallas_call(
        _body,
        out_shape=(
            jax.ShapeDtypeStruct((_B, _C, _N), jnp.float32),
            jax.ShapeDtypeStruct((1, _T), jnp.int32),
            jax.ShapeDtypeStruct((1, 1), jnp.float32),
            jax.ShapeDtypeStruct((1, 1), jnp.float32),
            jax.ShapeDtypeStruct((1, 1), jnp.float32),
            jax.ShapeDtypeStruct((1, 1), jnp.float32),
        ),
    )(x3, super_points, W_down, b_down, W_up, b_up)
    q = q3.reshape(_B, _C, _H, _W)
    idx_flat = idx.reshape(-1)
    return (q, idx_flat, em.reshape(()), me.reshape(()),
            el.reshape(()), cl.reshape(()))


# speedup vs baseline: 2.2781x; 1.0010x over previous
import jax
import jax.numpy as jnp
from jax.experimental import pallas as pl

_B, _C, _H, _W = 4, 192, 14, 14
_N = _H * _W
_T = _B * _N
_D = 32
_K = 1024


def _body(x_ref, sp_ref, wd_ref, bd_ref, wu_ref, bu_ref,
          q_ref, idx_ref, em_ref, me_ref, el_ref, cl_ref):
    wd = wd_ref[...]
    y = jnp.concatenate(
        [jnp.dot(wd, x_ref[b], preferred_element_type=jnp.float32)
         for b in range(_B)], axis=1)
    y = y + bd_ref[...].reshape(_D, 1)

    norms = jnp.sqrt(jnp.sum(y * y, axis=0, keepdims=True))
    xn = y / norms

    sp = sp_ref[...]
    dots = jnp.dot(sp, xn, precision=jax.lax.Precision.HIGHEST,
                   preferred_element_type=jnp.float32)
    dmax = jnp.max(dots, axis=0, keepdims=True)
    kiota = jax.lax.broadcasted_iota(jnp.int32, (_K, _T), 0)
    idx = jnp.min(jnp.where(dots == dmax, kiota, jnp.int32(1 << 30)),
                  axis=0, keepdims=True)
    idx_ref[...] = idx

    z2 = (dots - dmax) * (100.0 * 1.4426950408889634)
    ex = jnp.exp2(z2)
    sums = jnp.sum(ex, axis=0, keepdims=True)
    inv = 1.0 / sums
    sez = jnp.sum(ex * z2, axis=0, keepdims=True) * (1.0 / 1.4426950408889634)
    sample_entropy = jnp.log(sums) - inv * sez
    em = jnp.sum(sample_entropy, axis=1, keepdims=True) / float(_T)
    ap = jnp.sum(ex * inv, axis=1, keepdims=True) / float(_T)
    me = -jnp.sum(ap * jnp.log(ap + 1e-15), axis=0, keepdims=True)

    oh = (kiota == idx).astype(jnp.float32)
    quant = jax.lax.dot_general(
        sp, oh, (((0,), (0,)), ((), ())),
        preferred_element_type=jnp.float32)

    st = xn + (quant - xn)
    diff = xn - st
    cl = jnp.sum(jnp.sum(diff * diff, axis=0, keepdims=True),
                 axis=1, keepdims=True) / float(_D * _T)

    em_ref[...] = em
    me_ref[...] = me
    el_ref[...] = em - me
    cl_ref[...] = cl

    q = jnp.dot(wu_ref[...], st,
                preferred_element_type=jnp.float32)
    q = q + bu_ref[...].reshape(_C, 1)
    for b in range(_B):
        q_ref[b] = q[:, b * _N:(b + 1) * _N]


def kernel(x, super_points, W_down, b_down, W_up, b_up):
    x3 = x.reshape(_B, _C, _N)
    q3, idx, em, me, el, cl = pl.pallas_call(
        _body,
        out_shape=(
            jax.ShapeDtypeStruct((_B, _C, _N), jnp.float32),
            jax.ShapeDtypeStruct((1, _T), jnp.int32),
            jax.ShapeDtypeStruct((1, 1), jnp.float32),
            jax.ShapeDtypeStruct((1, 1), jnp.float32),
            jax.ShapeDtypeStruct((1, 1), jnp.float32),
            jax.ShapeDtypeStruct((1, 1), jnp.float32),
        ),
    )(x3, super_points, W_down, b_down, W_up, b_up)
    q = q3.reshape(_B, _C, _H, _W)
    idx_flat = idx.reshape(-1)
    return (q, idx_flat, em.reshape(()), me.reshape(()),
            el.reshape(()), cl.reshape(()))
